# Initial kernel scaffold; baseline (speedup 1.0000x reference)
import jax
import jax.numpy as jnp
from jax.experimental import pallas as pl


def kernel(points, W1, b1, g1, be1, W3, b3, g3, be3, W4, b4, g4, be4, W5, b5, g5, be5, W6, b6, g6, be6, W7, b7):
    raise NotImplementedError("write your pallas kernel here")


# speedup vs baseline: 18.3499x; 18.3499x over previous
import functools

import jax
import jax.numpy as jnp
from jax import lax
from jax.experimental import pallas as pl
from jax.experimental.pallas import tpu as pltpu
from jax.experimental.pallas import tpu_sc as plsc

B, N, K = 4, 4096, 8
BN = B * N
TN = 512
GRID = BN // TN
TR = 256
EPS = 1e-5
_BIG = 1 << 30
_INF = float("inf")


def _accum_stats(i, h, s_ref, ss_ref):
    ts = jnp.sum(h, axis=0, keepdims=True)
    tss = jnp.sum(h * h, axis=0, keepdims=True)

    @pl.when(i == 0)
    def _():
        s_ref[...] = ts
        ss_ref[...] = tss

    @pl.when(i > 0)
    def _():
        s_ref[...] += ts
        ss_ref[...] += tss


def _k1_body(pts_ref, w1t_ref, b1_ref, h1_ref, s_ref, ss_ref):
    i = pl.program_id(0)
    h = jnp.dot(pts_ref[...], w1t_ref[...],
                preferred_element_type=jnp.float32) + b1_ref[...]
    h1_ref[...] = h
    _accum_stats(i, h, s_ref, ss_ref)


def _k2_body(h1_ref, a1_ref, c1_ref, w3t_ref, b3_ref,
             lf_ref, h3low_ref, s_ref, ss_ref, hmax_ref, hmin_ref):
    i = pl.program_id(0)
    lf = jnp.maximum(h1_ref[...] * a1_ref[...] + c1_ref[...], 0.0)
    lf_ref[...] = lf
    h3 = jnp.dot(lf, w3t_ref[...],
                 preferred_element_type=jnp.float32) + b3_ref[...]
    h3low_ref[...] = h3[:, :K]
    _accum_stats(i, h3, s_ref, ss_ref)
    tmax = jnp.max(h3, axis=0, keepdims=True)
    tmin = jnp.min(h3, axis=0, keepdims=True)

    @pl.when(i % (GRID // B) == 0)
    def _():
        hmax_ref[0] = tmax
        hmin_ref[0] = tmin

    @pl.when(i % (GRID // B) > 0)
    def _():
        hmax_ref[0] = jnp.maximum(hmax_ref[0], tmax)
        hmin_ref[0] = jnp.minimum(hmin_ref[0], tmin)


def _k5_body(lf_ref, loc_ref, gmax_ref, w4mt_ref, w4gt_ref, w4s_ref, b4_ref,
             h4_ref, s_ref, ss_ref):
    i = pl.program_id(0)
    gterm = jnp.dot(gmax_ref[0], w4gt_ref[...],
                    preferred_element_type=jnp.float32)
    h = (jnp.dot(lf_ref[...], w4mt_ref[...],
                 preferred_element_type=jnp.float32)
         + loc_ref[...] * w4s_ref[...]
         + gterm + b4_ref[...])
    h4_ref[...] = h
    _accum_stats(i, h, s_ref, ss_ref)


def _mid_body(h_ref, a_ref, c_ref, wt_ref, b_ref, out_ref, s_ref, ss_ref):
    i = pl.program_id(0)
    f = jnp.maximum(h_ref[...] * a_ref[...] + c_ref[...], 0.0)
    h = jnp.dot(f, wt_ref[...],
                preferred_element_type=jnp.float32) + b_ref[...]
    out_ref[...] = h
    _accum_stats(i, h, s_ref, ss_ref)


def _k8_body(h_ref, a_ref, c_ref, wt_ref, b_ref, out_ref):
    f = jnp.maximum(h_ref[...] * a_ref[...] + c_ref[...], 0.0)
    out_ref[...] = jnp.dot(f, wt_ref[...],
                           preferred_element_type=jnp.float32) + b_ref[...]


def _row_spec(c):
    return pl.BlockSpec((TN, c), lambda i: (i, 0))


def _full_spec(r, c):
    return pl.BlockSpec((r, c), lambda i: (0, 0))


def _stat_spec(c):
    return pl.BlockSpec((1, c), lambda i: (0, 0))


def _batch_spec(c):
    return pl.BlockSpec((1, 1, c), lambda i: (i // (GRID // B), 0, 0))


def _f32(*shape):
    return jax.ShapeDtypeStruct(shape, jnp.float32)


def _knn_body(prow_ref, pcolt_ref, idx_ref):
    pr = prow_ref[0]
    pt = pcolt_ref[0]
    sqr = jnp.sum(pr * pr, axis=1, keepdims=True)
    sqc = jnp.sum(pt * pt, axis=0, keepdims=True)
    d = sqr + sqc - 2.0 * jnp.dot(pr, pt, preferred_element_type=jnp.float32)
    iota = lax.broadcasted_iota(jnp.int32, (TR, N), 1)
    cols = []
    for _ in range(K):
        mv = jnp.min(d, axis=1, keepdims=True)
        idxc = jnp.where(d == mv, iota, _BIG)
        m = jnp.min(idxc, axis=1, keepdims=True)
        cols.append(m)
        d = jnp.where(iota == m, _INF, d)
    idx_ref[0] = jnp.concatenate(cols, axis=1)


def _knn_topk(pts_pad, pts_t):
    return pl.pallas_call(
        _knn_body,
        grid=(B, N // TR),
        in_specs=[
            pl.BlockSpec((1, TR, 8), lambda b, r: (b, r, 0)),
            pl.BlockSpec((1, 8, N), lambda b, r: (b, 0, 0)),
        ],
        out_specs=pl.BlockSpec((1, TR, K), lambda b, r: (b, r, 0)),
        out_shape=jax.ShapeDtypeStruct((B, N, K), jnp.int32),
    )(pts_pad, pts_t)


_NW = 32
_PPW = BN // _NW
_STEPS = _PPW // 2


def _perm16(x, idx):
    return lax.gather(
        x, idx[:, None],
        lax.GatherDimensionNumbers(offset_dims=(), collapsed_slice_dims=(0,),
                                   start_index_map=(0,)),
        (1,), mode=lax.GatherScatterMode.PROMISE_IN_BOUNDS)


def _sc_local_max(h3low_flat, idx_flat, apat, cpat):
    mesh = plsc.VectorSubcoreMesh(core_axis_name="c", subcore_axis_name="s")

    @functools.partial(
        pl.kernel,
        out_type=jax.ShapeDtypeStruct((BN,), jnp.float32),
        mesh=mesh,
        compiler_params=pltpu.CompilerParams(needs_layout_passes=False),
        scratch_types=[
            pltpu.VMEM((N * K,), jnp.float32),
            pltpu.VMEM((_PPW * K,), jnp.int32),
            pltpu.VMEM((16,), jnp.float32),
            pltpu.VMEM((16,), jnp.float32),
            pltpu.VMEM((_PPW,), jnp.float32),
        ],
    )
    def k(h3_hbm, idx_hbm, ap_hbm, cp_hbm, out_hbm, h3_v, idx_v, ap_v, cp_v,
          loc_v):
        wid = lax.axis_index("s") * 2 + lax.axis_index("c")
        b = wid // (_NW // B)
        base = wid * _PPW
        pltpu.sync_copy(h3_hbm.at[pl.ds(b * (N * K), N * K)], h3_v)
        pltpu.sync_copy(idx_hbm.at[pl.ds(base * K, _PPW * K)], idx_v)
        pltpu.sync_copy(ap_hbm, ap_v)
        pltpu.sync_copy(cp_hbm, cp_v)
        ap = ap_v[...]
        cp = cp_v[...]
        lane = lax.iota(jnp.int32, 16)
        jlane = lane & 7
        p1 = lane ^ 1
        p2 = lane ^ 2
        p4 = lane ^ 4
        dmask = jlane == 0

        def body(s, carry):
            ridx = idx_v[pl.ds(s * 16, 16)]
            g = plsc.load_gather(h3_v, [ridx * K + jlane])
            xv = jnp.maximum(g * ap + cp, 0.0)
            xv = jnp.maximum(xv, _perm16(xv, p1))
            xv = jnp.maximum(xv, _perm16(xv, p2))
            xv = jnp.maximum(xv, _perm16(xv, p4))
            dest = jnp.where(lane < 8, 2 * s, 2 * s + 1)
            plsc.store_scatter(loc_v, [dest], xv, mask=dmask)
            return carry

        lax.fori_loop(0, _STEPS, body, 0)
        pltpu.sync_copy(loc_v, out_hbm.at[pl.ds(base, _PPW)])

    return k(h3low_flat, idx_flat, apat, cpat)


def _bn_affine(s, ss, gamma, beta):
    mean = s / BN
    var = ss / BN - mean * mean
    a = gamma[None, :] / jnp.sqrt(var + EPS)
    c = beta[None, :] - mean * a
    return a, c


def kernel(points, W1, b1, g1, be1, W3, b3, g3, be3, W4, b4, g4, be4,
           W5, b5, g5, be5, W6, b6, g6, be6, W7, b7):
    f32 = jnp.float32
    pts = points.reshape(BN, 3).astype(f32)
    pts_pad8 = jnp.pad(pts, ((0, 0), (0, 5)))
    pts_pad = pts_pad8.reshape(B, N, 8)
    pts_t = jnp.transpose(pts_pad, (0, 2, 1))

    w1t = W1.T
    w3t = W3.T
    w4mt = W4[:, 512:576].T
    w4gt = W4[:, 576:].T
    w4s = jnp.sum(
        lax.optimization_barrier(
            W4[:, :512].astype(jnp.bfloat16)).astype(f32),
        axis=1)[None, :]
    w5t = W5.T
    w6t = W6.T
    w7t = W7.T
    r2 = lambda v: v[None, :]

    h1, s1, ss1 = pl.pallas_call(
        _k1_body,
        grid=(GRID,),
        in_specs=[_row_spec(8), _full_spec(8, 64), _stat_spec(64)],
        out_specs=[_row_spec(64), _stat_spec(64), _stat_spec(64)],
        out_shape=[_f32(BN, 64), _f32(1, 64), _f32(1, 64)],
    )(pts_pad8, jnp.pad(w1t, ((0, 5), (0, 0))), r2(b1))
    a1, c1 = _bn_affine(s1, ss1, g1, be1)

    lf, h3low, s3, ss3, hmax3, hmin3 = pl.pallas_call(
        _k2_body,
        grid=(GRID,),
        in_specs=[_row_spec(64), _stat_spec(64), _stat_spec(64),
                  _full_spec(64, 512), _stat_spec(512)],
        out_specs=[_row_spec(64), _row_spec(K), _stat_spec(512),
                   _stat_spec(512), _batch_spec(512), _batch_spec(512)],
        out_shape=[_f32(BN, 64), _f32(BN, K), _f32(1, 512), _f32(1, 512),
                   _f32(B, 1, 512), _f32(B, 1, 512)],
    )(h1, a1, c1, w3t, r2(b3))
    a3, c3 = _bn_affine(s3, ss3, g3, be3)

    idx8 = _knn_topk(pts_pad, pts_t)

    apat = jnp.tile(a3[0, :K], 2)
    cpat = jnp.tile(c3[0, :K], 2)
    local = _sc_local_max(h3low.reshape(BN * K), idx8.reshape(BN * K),
                          apat, cpat)
    local = lax.optimization_barrier(
        local.astype(jnp.bfloat16)).astype(f32).reshape(BN, 1)

    gmax = jnp.maximum(
        jnp.maximum(a3[None] * hmax3, a3[None] * hmin3) + c3[None],
        0.0)

    h4, s4, ss4 = pl.pallas_call(
        _k5_body,
        grid=(GRID,),
        in_specs=[_row_spec(64), _row_spec(1), _batch_spec(512),
                  _full_spec(64, 512), _full_spec(512, 512),
                  _stat_spec(512), _stat_spec(512)],
        out_specs=[_row_spec(512), _stat_spec(512), _stat_spec(512)],
        out_shape=[_f32(BN, 512), _f32(1, 512), _f32(1, 512)],
    )(lf, local, gmax, w4mt, w4gt, w4s, r2(b4))
    a4, c4 = _bn_affine(s4, ss4, g4, be4)

    h5, s5, ss5 = pl.pallas_call(
        _mid_body,
        grid=(GRID,),
        in_specs=[_row_spec(512), _stat_spec(512), _stat_spec(512),
                  _full_spec(512, 256), _stat_spec(256)],
        out_specs=[_row_spec(256), _stat_spec(256), _stat_spec(256)],
        out_shape=[_f32(BN, 256), _f32(1, 256), _f32(1, 256)],
    )(h4, a4, c4, w5t, r2(b5))
    a5, c5 = _bn_affine(s5, ss5, g5, be5)

    h6, s6, ss6 = pl.pallas_call(
        _mid_body,
        grid=(GRID,),
        in_specs=[_row_spec(256), _stat_spec(256), _stat_spec(256),
                  _full_spec(256, 128), _stat_spec(128)],
        out_specs=[_row_spec(128), _stat_spec(128), _stat_spec(128)],
        out_shape=[_f32(BN, 128), _f32(1, 128), _f32(1, 128)],
    )(h5, a5, c5, w6t, r2(b6))
    a6, c6 = _bn_affine(s6, ss6, g6, be6)

    out = pl.pallas_call(
        _k8_body,
        grid=(GRID,),
        in_specs=[_row_spec(128), _stat_spec(128), _stat_spec(128),
                  _full_spec(128, 6), _stat_spec(6)],
        out_specs=_row_spec(6),
        out_shape=_f32(BN, 6),
    )(h6, a6, c6, w7t, r2(b7))
    return out.reshape(B, N, 6)


# speedup vs baseline: 20.8647x; 1.1370x over previous
import functools

import jax
import jax.numpy as jnp
from jax import lax
from jax.experimental import pallas as pl
from jax.experimental.pallas import tpu as pltpu
from jax.experimental.pallas import tpu_sc as plsc

B, N, K = 4, 4096, 8
BN = B * N
TN = 512
GRID = BN // TN
TR = 1024
EPS = 1e-5
_BIG = 1 << 30
_INF = float("inf")


def _accum_stats(i, h, s_ref, ss_ref):
    ts = jnp.sum(h, axis=0, keepdims=True)
    tss = jnp.sum(h * h, axis=0, keepdims=True)

    @pl.when(i == 0)
    def _():
        s_ref[...] = ts
        ss_ref[...] = tss

    @pl.when(i > 0)
    def _():
        s_ref[...] += ts
        ss_ref[...] += tss


def _k1_body(pts_ref, w1t_ref, b1_ref, h1_ref, s_ref, ss_ref):
    i = pl.program_id(0)
    h = jnp.dot(pts_ref[...], w1t_ref[...],
                preferred_element_type=jnp.float32) + b1_ref[...]
    h1_ref[...] = h
    _accum_stats(i, h, s_ref, ss_ref)


def _k2_body(h1_ref, a1_ref, c1_ref, w3t_ref, b3_ref,
             lf_ref, h3low_ref, s_ref, ss_ref, hmax_ref, hmin_ref):
    i = pl.program_id(0)
    lf = jnp.maximum(h1_ref[...] * a1_ref[...] + c1_ref[...], 0.0)
    lf_ref[...] = lf
    h3 = jnp.dot(lf, w3t_ref[...],
                 preferred_element_type=jnp.float32) + b3_ref[...]
    h3low_ref[...] = h3[:, :K]
    _accum_stats(i, h3, s_ref, ss_ref)
    tmax = jnp.max(h3, axis=0, keepdims=True)
    tmin = jnp.min(h3, axis=0, keepdims=True)

    @pl.when(i % (GRID // B) == 0)
    def _():
        hmax_ref[0] = tmax
        hmin_ref[0] = tmin

    @pl.when(i % (GRID // B) > 0)
    def _():
        hmax_ref[0] = jnp.maximum(hmax_ref[0], tmax)
        hmin_ref[0] = jnp.minimum(hmin_ref[0], tmin)


def _k5_body(lf_ref, loc_ref, gmax_ref, w4mt_ref, w4gt_ref, w4s_ref, b4_ref,
             h4_ref, s_ref, ss_ref):
    i = pl.program_id(0)
    gterm = jnp.dot(gmax_ref[0], w4gt_ref[...],
                    preferred_element_type=jnp.float32)
    h = (jnp.dot(lf_ref[...], w4mt_ref[...],
                 preferred_element_type=jnp.float32)
         + loc_ref[...] * w4s_ref[...]
         + gterm + b4_ref[...])
    h4_ref[...] = h
    _accum_stats(i, h, s_ref, ss_ref)


def _mid_body(h_ref, a_ref, c_ref, wt_ref, b_ref, out_ref, s_ref, ss_ref):
    i = pl.program_id(0)
    f = jnp.maximum(h_ref[...] * a_ref[...] + c_ref[...], 0.0)
    h = jnp.dot(f, wt_ref[...],
                preferred_element_type=jnp.float32) + b_ref[...]
    out_ref[...] = h
    _accum_stats(i, h, s_ref, ss_ref)


def _k8_body(h_ref, a_ref, c_ref, wt_ref, b_ref, out_ref):
    f = jnp.maximum(h_ref[...] * a_ref[...] + c_ref[...], 0.0)
    out_ref[...] = jnp.dot(f, wt_ref[...],
                           preferred_element_type=jnp.float32) + b_ref[...]


def _row_spec(c):
    return pl.BlockSpec((TN, c), lambda i: (i, 0))


def _full_spec(r, c):
    return pl.BlockSpec((r, c), lambda i: (0, 0))


def _stat_spec(c):
    return pl.BlockSpec((1, c), lambda i: (0, 0))


def _batch_spec(c):
    return pl.BlockSpec((1, 1, c), lambda i: (i // (GRID // B), 0, 0))


def _f32(*shape):
    return jax.ShapeDtypeStruct(shape, jnp.float32)


def _knn_body(prow_ref, pcolt_ref, idx_ref):
    pr = prow_ref[0]
    pt = pcolt_ref[0]
    sqr = jnp.sum(pr * pr, axis=1, keepdims=True)
    sqc = jnp.sum(pt * pt, axis=0, keepdims=True)
    d = sqr + sqc - 2.0 * jnp.dot(pr, pt, preferred_element_type=jnp.float32)
    iota = lax.broadcasted_iota(jnp.int32, (TR, N), 1)
    cols = []
    for _ in range(K):
        m = jnp.argmin(d, axis=1).astype(jnp.int32)[:, None]
        cols.append(m)
        d = jnp.where(iota == m, _INF, d)
    idx_ref[0] = jnp.concatenate(cols, axis=1)


def _knn_topk(pts_pad, pts_t):
    return pl.pallas_call(
        _knn_body,
        grid=(B, N // TR),
        in_specs=[
            pl.BlockSpec((1, TR, 8), lambda b, r: (b, r, 0)),
            pl.BlockSpec((1, 8, N), lambda b, r: (b, 0, 0)),
        ],
        out_specs=pl.BlockSpec((1, TR, K), lambda b, r: (b, r, 0)),
        out_shape=jax.ShapeDtypeStruct((B, N, K), jnp.int32),
    )(pts_pad, pts_t)


_NW = 32
_PPW = BN // _NW
_STEPS = _PPW // 2


def _perm16(x, idx):
    return lax.gather(
        x, idx[:, None],
        lax.GatherDimensionNumbers(offset_dims=(), collapsed_slice_dims=(0,),
                                   start_index_map=(0,)),
        (1,), mode=lax.GatherScatterMode.PROMISE_IN_BOUNDS)


def _sc_local_max(h3low_flat, idx_flat, apat, cpat):
    mesh = plsc.VectorSubcoreMesh(core_axis_name="c", subcore_axis_name="s")

    @functools.partial(
        pl.kernel,
        out_type=jax.ShapeDtypeStruct((BN,), jnp.float32),
        mesh=mesh,
        compiler_params=pltpu.CompilerParams(needs_layout_passes=False),
        scratch_types=[
            pltpu.VMEM((N * K,), jnp.float32),
            pltpu.VMEM((_PPW * K,), jnp.int32),
            pltpu.VMEM((16,), jnp.float32),
            pltpu.VMEM((16,), jnp.float32),
            pltpu.VMEM((_PPW,), jnp.float32),
        ],
    )
    def k(h3_hbm, idx_hbm, ap_hbm, cp_hbm, out_hbm, h3_v, idx_v, ap_v, cp_v,
          loc_v):
        wid = lax.axis_index("s") * 2 + lax.axis_index("c")
        b = wid // (_NW // B)
        base = wid * _PPW
        pltpu.sync_copy(h3_hbm.at[pl.ds(b * (N * K), N * K)], h3_v)
        pltpu.sync_copy(idx_hbm.at[pl.ds(base * K, _PPW * K)], idx_v)
        pltpu.sync_copy(ap_hbm, ap_v)
        pltpu.sync_copy(cp_hbm, cp_v)
        ap = ap_v[...]
        cp = cp_v[...]
        lane = lax.iota(jnp.int32, 16)
        jlane = lane & 7
        p1 = lane ^ 1
        p2 = lane ^ 2
        p4 = lane ^ 4
        dmask = jlane == 0

        def body(s, carry):
            ridx = idx_v[pl.ds(s * 16, 16)]
            g = plsc.load_gather(h3_v, [ridx * K + jlane])
            xv = jnp.maximum(g * ap + cp, 0.0)
            xv = jnp.maximum(xv, _perm16(xv, p1))
            xv = jnp.maximum(xv, _perm16(xv, p2))
            xv = jnp.maximum(xv, _perm16(xv, p4))
            dest = jnp.where(lane < 8, 2 * s, 2 * s + 1)
            plsc.store_scatter(loc_v, [dest], xv, mask=dmask)
            return carry

        lax.fori_loop(0, _STEPS, body, 0)
        pltpu.sync_copy(loc_v, out_hbm.at[pl.ds(base, _PPW)])

    return k(h3low_flat, idx_flat, apat, cpat)


def _bn_affine(s, ss, gamma, beta):
    mean = s / BN
    var = ss / BN - mean * mean
    a = gamma[None, :] / jnp.sqrt(var + EPS)
    c = beta[None, :] - mean * a
    return a, c


def kernel(points, W1, b1, g1, be1, W3, b3, g3, be3, W4, b4, g4, be4,
           W5, b5, g5, be5, W6, b6, g6, be6, W7, b7):
    f32 = jnp.float32
    pts = points.reshape(BN, 3).astype(f32)
    pts_pad8 = jnp.pad(pts, ((0, 0), (0, 5)))
    pts_pad = pts_pad8.reshape(B, N, 8)
    pts_t = jnp.transpose(pts_pad, (0, 2, 1))

    w1t = W1.T
    w3t = W3.T
    w4mt = W4[:, 512:576].T
    w4gt = W4[:, 576:].T
    w4s = jnp.sum(
        lax.optimization_barrier(
            W4[:, :512].astype(jnp.bfloat16)).astype(f32),
        axis=1)[None, :]
    w5t = W5.T
    w6t = W6.T
    w7t = W7.T
    r2 = lambda v: v[None, :]

    h1, s1, ss1 = pl.pallas_call(
        _k1_body,
        grid=(GRID,),
        in_specs=[_row_spec(8), _full_spec(8, 64), _stat_spec(64)],
        out_specs=[_row_spec(64), _stat_spec(64), _stat_spec(64)],
        out_shape=[_f32(BN, 64), _f32(1, 64), _f32(1, 64)],
    )(pts_pad8, jnp.pad(w1t, ((0, 5), (0, 0))), r2(b1))
    a1, c1 = _bn_affine(s1, ss1, g1, be1)

    lf, h3low, s3, ss3, hmax3, hmin3 = pl.pallas_call(
        _k2_body,
        grid=(GRID,),
        in_specs=[_row_spec(64), _stat_spec(64), _stat_spec(64),
                  _full_spec(64, 512), _stat_spec(512)],
        out_specs=[_row_spec(64), _row_spec(K), _stat_spec(512),
                   _stat_spec(512), _batch_spec(512), _batch_spec(512)],
        out_shape=[_f32(BN, 64), _f32(BN, K), _f32(1, 512), _f32(1, 512),
                   _f32(B, 1, 512), _f32(B, 1, 512)],
    )(h1, a1, c1, w3t, r2(b3))
    a3, c3 = _bn_affine(s3, ss3, g3, be3)

    idx8 = _knn_topk(pts_pad, pts_t)

    apat = jnp.tile(a3[0, :K], 2)
    cpat = jnp.tile(c3[0, :K], 2)
    local = _sc_local_max(h3low.reshape(BN * K), idx8.reshape(BN * K),
                          apat, cpat)
    local = lax.optimization_barrier(
        local.astype(jnp.bfloat16)).astype(f32).reshape(BN, 1)

    gmax = jnp.maximum(
        jnp.maximum(a3[None] * hmax3, a3[None] * hmin3) + c3[None],
        0.0)

    h4, s4, ss4 = pl.pallas_call(
        _k5_body,
        grid=(GRID,),
        in_specs=[_row_spec(64), _row_spec(1), _batch_spec(512),
                  _full_spec(64, 512), _full_spec(512, 512),
                  _stat_spec(512), _stat_spec(512)],
        out_specs=[_row_spec(512), _stat_spec(512), _stat_spec(512)],
        out_shape=[_f32(BN, 512), _f32(1, 512), _f32(1, 512)],
    )(lf, local, gmax, w4mt, w4gt, w4s, r2(b4))
    a4, c4 = _bn_affine(s4, ss4, g4, be4)

    h5, s5, ss5 = pl.pallas_call(
        _mid_body,
        grid=(GRID,),
        in_specs=[_row_spec(512), _stat_spec(512), _stat_spec(512),
                  _full_spec(512, 256), _stat_spec(256)],
        out_specs=[_row_spec(256), _stat_spec(256), _stat_spec(256)],
        out_shape=[_f32(BN, 256), _f32(1, 256), _f32(1, 256)],
    )(h4, a4, c4, w5t, r2(b5))
    a5, c5 = _bn_affine(s5, ss5, g5, be5)

    h6, s6, ss6 = pl.pallas_call(
        _mid_body,
        grid=(GRID,),
        in_specs=[_row_spec(256), _stat_spec(256), _stat_spec(256),
                  _full_spec(256, 128), _stat_spec(128)],
        out_specs=[_row_spec(128), _stat_spec(128), _stat_spec(128)],
        out_shape=[_f32(BN, 128), _f32(1, 128), _f32(1, 128)],
    )(h5, a5, c5, w6t, r2(b6))
    a6, c6 = _bn_affine(s6, ss6, g6, be6)

    out = pl.pallas_call(
        _k8_body,
        grid=(GRID,),
        in_specs=[_row_spec(128), _stat_spec(128), _stat_spec(128),
                  _full_spec(128, 6), _stat_spec(6)],
        out_specs=_row_spec(6),
        out_shape=_f32(BN, 6),
    )(h6, a6, c6, w7t, r2(b7))
    return out.reshape(B, N, 6)
